# baseline (speedup 1.0000x reference)
import jax
import jax.numpy as jnp
from jax import lax
from jax.experimental import pallas as pl
from jax.experimental.pallas import tpu as pltpu
from jax.experimental.pallas import tpu_sc as plsc

_N = 320000
_T = 10000
_D = 128
_BLK = 64
_NB = _N // _BLK
_TP = 10112
_RPT = _TP // 16

_MAGIC = 0x5F3759DF


def _sc_body(raw_hbm, idx_hbm, out_tab, out_cnt,
             idxr, buf0, buf1, buf2, buf3, cnt, table,
             sem_r0, sem_r1, sem_r2, sem_r3, sem_ji, sem_s):
    c = lax.axis_index("c")
    s = lax.axis_index("s")
    wid = c * 16 + s
    zero16 = jnp.zeros((16,), jnp.float32)
    one16 = jnp.full((16,), 1.0, jnp.float32)
    bufs = [buf0, buf1, buf2, buf3]
    sems = [sem_r0, sem_r1, sem_r2, sem_r3]

    def zrow(r, _):
        for k in range(8):
            buf0[r, pl.ds(k * 16, 16)] = zero16
        return 0

    lax.fori_loop(0, _BLK, zrow, 0)
    base_t = s * _RPT
    for t in range(9):
        pltpu.sync_copy(buf0.at[pl.ds(0, 64)],
                        table.at[pl.ds(base_t + t * 64, 64)])
    pltpu.sync_copy(buf0.at[pl.ds(0, 56)],
                    table.at[pl.ds(base_t + 576, 56)])

    def crow(r, _):
        cnt[pl.ds(r * 16, 16)] = zero16
        return 0

    lax.fori_loop(0, _TP // 16, crow, 0)
    plsc.subcore_barrier()

    start_blk = 156 * wid + jnp.minimum(wid, 8)

    def norm_block(buf):
        def group16(g, _):
            for r in range(16):
                row = g * 16 + r
                vs = [buf[row, pl.ds(k * 16, 16)] for k in range(8)]
                acc = vs[0] * vs[0]
                for k in range(1, 8):
                    acc = acc + vs[k] * vs[k]
                cs = plsc.cumsum(acc)
                x = cs[jnp.full((16,), 15, jnp.int32)]
                i = _MAGIC - lax.shift_right_logical(plsc.bitcast(x, jnp.int32), 1)
                y = plsc.bitcast(i, jnp.float32)
                for _ in range(3):
                    y = y * (1.5 - 0.5 * x * y * y)
                for k in range(8):
                    buf[row, pl.ds(k * 16, 16)] = vs[k] * y
            return 0

        lax.fori_loop(0, _BLK // 16, group16, 0)

    def count_block(krow):
        for g in range(_BLK // 16):
            idv = idxr[krow, pl.ds(g * 16, 16)]
            plsc.addupdate_scatter(cnt, [idv], one16)

    def quad(q, _):
        base = start_blk + 4 * q

        @pl.when(q > 0)
        def _drain():
            for k in range(4):
                pltpu.make_async_copy(bufs[k], table.at[idxr.at[k]],
                                      sem_s).wait()

        icp = pltpu.async_copy(idx_hbm.at[pl.ds(base, 4)], idxr, sem_ji)
        incp = [pltpu.async_copy(raw_hbm.at[pl.ds((base + k) * _BLK, _BLK)],
                                 bufs[k], sems[k]) for k in range(4)]
        icp.wait()
        for k in range(4):
            incp[k].wait()
            norm_block(bufs[k])
            count_block(k)
            pltpu.async_copy(bufs[k], table.at[idxr.at[k]], sem_s, add=True)
        return 0

    lax.fori_loop(0, 39, quad, 0)
    for k in range(4):
        pltpu.make_async_copy(bufs[k], table.at[idxr.at[k]], sem_s).wait()

    @pl.when(wid < 8)
    def _tail():
        b = start_blk + 156
        pltpu.sync_copy(idx_hbm.at[pl.ds(b, 1)], idxr.at[pl.ds(0, 1)])
        pltpu.sync_copy(raw_hbm.at[pl.ds(b * _BLK, _BLK)], buf0)
        norm_block(buf0)
        count_block(0)
        pltpu.sync_copy(buf0, table.at[idxr.at[0]], add=True)

    pltpu.sync_copy(cnt, out_cnt.at[wid])
    plsc.subcore_barrier()
    pltpu.sync_copy(table.at[pl.ds(base_t, _RPT)],
                    out_tab.at[c, pl.ds(base_t, _RPT)])


_sc_scatter = pl.kernel(
    _sc_body,
    out_type=(jax.ShapeDtypeStruct((2, _TP, _D), jnp.float32),
              jax.ShapeDtypeStruct((32, _TP), jnp.float32)),
    mesh=plsc.VectorSubcoreMesh(core_axis_name="c", subcore_axis_name="s"),
    compiler_params=pltpu.CompilerParams(use_tc_tiling_on_sc=False,
                                         needs_layout_passes=False),
    scratch_types=[
        pltpu.VMEM((4, _BLK), jnp.int32),
        pltpu.VMEM((_BLK, _D), jnp.float32),
        pltpu.VMEM((_BLK, _D), jnp.float32),
        pltpu.VMEM((_BLK, _D), jnp.float32),
        pltpu.VMEM((_BLK, _D), jnp.float32),
        pltpu.VMEM((_TP,), jnp.float32),
        pltpu.VMEM_SHARED((_TP, _D), jnp.float32),
        pltpu.SemaphoreType.DMA,
        pltpu.SemaphoreType.DMA,
        pltpu.SemaphoreType.DMA,
        pltpu.SemaphoreType.DMA,
        pltpu.SemaphoreType.DMA,
        pltpu.SemaphoreType.DMA,
    ],
)


def _epilogue_body(sp_ref, t_ref, c_ref, o_ref):
    sp = sp_ref[...]
    t = t_ref[0] + t_ref[1]
    seg_sum = t[:_T, :]
    counts = jnp.sum(c_ref[...], axis=0)[:_T, None]

    ss = jnp.sum(sp * sp, axis=1, keepdims=True)
    spn = sp / jnp.maximum(jnp.sqrt(ss), 1e-12)

    cc = jnp.maximum(counts, 1.0)
    mean = seg_sum / cc

    dot = jnp.sum(spn * mean, axis=1)
    na = jnp.maximum(jnp.sqrt(jnp.sum(spn * spn, axis=1)), 1e-8)
    nb = jnp.maximum(jnp.sqrt(jnp.sum(mean * mean, axis=1)), 1e-8)
    cos = dot / (na * nb)
    weights = counts[:, 0] / float(_N)
    o_ref[...] = jnp.sum((1.0 - cos) * weights).reshape(1, 1)


def _epilogue_tc(sp, tables, counts):
    return pl.pallas_call(
        _epilogue_body,
        out_shape=jax.ShapeDtypeStruct((1, 1), jnp.float32),
    )(sp, tables, counts)


def kernel(superPoint_feat, rawPoint_feat, point_assignment):
    idx = point_assignment.reshape(_NB, _BLK)
    tables, counts = _sc_scatter(rawPoint_feat, idx)
    loss = _epilogue_tc(superPoint_feat, tables, counts)
    return loss[0, 0]


# speedup vs baseline: 1.0488x; 1.0488x over previous
import jax
import jax.numpy as jnp
from jax import lax
from jax.experimental import pallas as pl
from jax.experimental.pallas import tpu as pltpu
from jax.experimental.pallas import tpu_sc as plsc

_N = 320000
_T = 10000
_D = 128
_BLK = 64
_NB = _N // _BLK
_TP = 10112
_RPT = _TP // 16

_MAGIC = 0x5F3759DF


def _sc_body(raw_hbm, idx_hbm, out_tab, out_cnt,
             idxr, buf0, buf1, buf2, buf3, cnt, table,
             sem_r0, sem_r1, sem_r2, sem_r3, sem_ji, sem_s):
    c = lax.axis_index("c")
    s = lax.axis_index("s")
    wid = c * 16 + s
    zero16 = jnp.zeros((16,), jnp.float32)
    one16 = jnp.full((16,), 1.0, jnp.float32)
    bufs = [buf0, buf1, buf2, buf3]
    sems = [sem_r0, sem_r1, sem_r2, sem_r3]

    def zrow(r, _):
        for k in range(8):
            buf0[r, pl.ds(k * 16, 16)] = zero16
        return 0

    lax.fori_loop(0, _BLK, zrow, 0)
    base_t = s * _RPT
    zcps = [pltpu.async_copy(buf0.at[pl.ds(0, 64)],
                             table.at[pl.ds(base_t + t * 64, 64)], sem_s)
            for t in range(9)]
    zcps.append(pltpu.async_copy(buf0.at[pl.ds(0, 56)],
                                 table.at[pl.ds(base_t + 576, 56)], sem_s))

    def crow(r, _):
        cnt[pl.ds(r * 16, 16)] = zero16
        return 0

    lax.fori_loop(0, _TP // 16, crow, 0)
    for cp in zcps:
        cp.wait()
    plsc.subcore_barrier()

    start_blk = 156 * wid + jnp.minimum(wid, 8)

    def norm_block(buf):
        def group16(g, _):
            for r in range(16):
                row = g * 16 + r
                vs = [buf[row, pl.ds(k * 16, 16)] for k in range(8)]
                acc = vs[0] * vs[0]
                for k in range(1, 8):
                    acc = acc + vs[k] * vs[k]
                cs = plsc.cumsum(acc)
                x = cs[jnp.full((16,), 15, jnp.int32)]
                i = _MAGIC - lax.shift_right_logical(plsc.bitcast(x, jnp.int32), 1)
                y = plsc.bitcast(i, jnp.float32)
                for _ in range(2):
                    y = y * (1.5 - 0.5 * x * y * y)
                for k in range(8):
                    buf[row, pl.ds(k * 16, 16)] = vs[k] * y
            return 0

        lax.fori_loop(0, _BLK // 16, group16, 0)

    def count_block(krow):
        for g in range(_BLK // 16):
            idv = idxr[krow, pl.ds(g * 16, 16)]
            plsc.addupdate_scatter(cnt, [idv], one16)

    def quad(q, _):
        base = start_blk + 4 * q

        @pl.when(q > 0)
        def _drain():
            for k in range(4):
                pltpu.make_async_copy(bufs[k], table.at[idxr.at[k]],
                                      sem_s).wait()

        icp = pltpu.async_copy(idx_hbm.at[pl.ds(base, 4)], idxr, sem_ji)
        incp = [pltpu.async_copy(raw_hbm.at[pl.ds((base + k) * _BLK, _BLK)],
                                 bufs[k], sems[k]) for k in range(4)]
        icp.wait()
        for k in range(4):
            incp[k].wait()
            norm_block(bufs[k])
            count_block(k)
            pltpu.async_copy(bufs[k], table.at[idxr.at[k]], sem_s, add=True)
        return 0

    lax.fori_loop(0, 39, quad, 0)
    for k in range(4):
        pltpu.make_async_copy(bufs[k], table.at[idxr.at[k]], sem_s).wait()

    @pl.when(wid < 8)
    def _tail():
        b = start_blk + 156
        pltpu.sync_copy(idx_hbm.at[pl.ds(b, 1)], idxr.at[pl.ds(0, 1)])
        pltpu.sync_copy(raw_hbm.at[pl.ds(b * _BLK, _BLK)], buf0)
        norm_block(buf0)
        count_block(0)
        pltpu.sync_copy(buf0, table.at[idxr.at[0]], add=True)

    pltpu.sync_copy(cnt, out_cnt.at[wid])
    plsc.subcore_barrier()
    pltpu.sync_copy(table.at[pl.ds(base_t, _RPT)],
                    out_tab.at[c, pl.ds(base_t, _RPT)])


_sc_scatter = pl.kernel(
    _sc_body,
    out_type=(jax.ShapeDtypeStruct((2, _TP, _D), jnp.float32),
              jax.ShapeDtypeStruct((32, _TP), jnp.float32)),
    mesh=plsc.VectorSubcoreMesh(core_axis_name="c", subcore_axis_name="s"),
    compiler_params=pltpu.CompilerParams(use_tc_tiling_on_sc=False,
                                         needs_layout_passes=False),
    scratch_types=[
        pltpu.VMEM((4, _BLK), jnp.int32),
        pltpu.VMEM((_BLK, _D), jnp.float32),
        pltpu.VMEM((_BLK, _D), jnp.float32),
        pltpu.VMEM((_BLK, _D), jnp.float32),
        pltpu.VMEM((_BLK, _D), jnp.float32),
        pltpu.VMEM((_TP,), jnp.float32),
        pltpu.VMEM_SHARED((_TP, _D), jnp.float32),
        pltpu.SemaphoreType.DMA,
        pltpu.SemaphoreType.DMA,
        pltpu.SemaphoreType.DMA,
        pltpu.SemaphoreType.DMA,
        pltpu.SemaphoreType.DMA,
        pltpu.SemaphoreType.DMA,
    ],
)


def _epilogue_body(sp_ref, t_ref, c_ref, o_ref):
    sp = sp_ref[...]
    t = t_ref[0] + t_ref[1]
    seg_sum = t[:_T, :]
    counts = jnp.sum(c_ref[...], axis=0)[:_T, None]

    ss = jnp.sum(sp * sp, axis=1, keepdims=True)
    spn = sp / jnp.maximum(jnp.sqrt(ss), 1e-12)

    cc = jnp.maximum(counts, 1.0)
    mean = seg_sum / cc

    dot = jnp.sum(spn * mean, axis=1)
    na = jnp.maximum(jnp.sqrt(jnp.sum(spn * spn, axis=1)), 1e-8)
    nb = jnp.maximum(jnp.sqrt(jnp.sum(mean * mean, axis=1)), 1e-8)
    cos = dot / (na * nb)
    weights = counts[:, 0] / float(_N)
    o_ref[...] = jnp.sum((1.0 - cos) * weights).reshape(1, 1)


def _epilogue_tc(sp, tables, counts):
    return pl.pallas_call(
        _epilogue_body,
        out_shape=jax.ShapeDtypeStruct((1, 1), jnp.float32),
    )(sp, tables, counts)


def kernel(superPoint_feat, rawPoint_feat, point_assignment):
    idx = point_assignment.reshape(_NB, _BLK)
    tables, counts = _sc_scatter(rawPoint_feat, idx)
    loss = _epilogue_tc(superPoint_feat, tables, counts)
    return loss[0, 0]
